# Initial kernel scaffold; baseline (speedup 1.0000x reference)
import jax
import jax.numpy as jnp
from jax.experimental import pallas as pl


def kernel(x, edge_index, W1, b1, W2, b2):
    raise NotImplementedError("write your pallas kernel here")


# speedup vs baseline: 4.8320x; 4.8320x over previous
import functools

import jax
import jax.numpy as jnp
from jax import lax
from jax.experimental import pallas as pl
from jax.experimental.pallas import tpu as pltpu
from jax.experimental.pallas import tpu_sc as plsc

N = 10000
E = 320000
D = 128
THRESH = 0.1
NC = 2
NS = 16
NW = NC * NS
EPW = E // NW
CH = 80
NCH = EPW // CH
NPAD = 10240
ZB = NPAD // NS

_mesh = plsc.VectorSubcoreMesh(core_axis_name="c", subcore_axis_name="s")


def _zero_vec(ref, nwords):
    def body(i, _):
        ref[pl.ds(i * 16, 16)] = jnp.zeros((16,), jnp.float32)
        return 0
    lax.fori_loop(0, nwords // 16, body, 0)


@functools.partial(
    pl.kernel,
    out_type=(
        jax.ShapeDtypeStruct((NW, NCH, CH), jnp.float32),
        jax.ShapeDtypeStruct((NC, NPAD), jnp.float32),
    ),
    mesh=_mesh,
    compiler_params=pltpu.CompilerParams(needs_layout_passes=False),
    scratch_types=(
        pltpu.VMEM((CH,), jnp.int32),
        pltpu.VMEM((CH,), jnp.int32),
        pltpu.VMEM((CH,), jnp.float32),
        pltpu.VMEM((CH, D), jnp.float32),
        pltpu.VMEM((CH, D), jnp.float32),
        pltpu.VMEM((ZB,), jnp.float32),
        pltpu.VMEM_SHARED((NPAD,), jnp.float32),
        pltpu.SemaphoreType.DMA,
        pltpu.SemaphoreType.DMA,
    ),
)
def _sc_attention(xn, row3, col3, att_out, rs_out, ridx, cidx, attv,
                  arows, brows, zbuf, rssh, sem1, sem2):
    c = lax.axis_index("c")
    s = lax.axis_index("s")
    w = s * NC + c

    _zero_vec(zbuf, ZB)
    pltpu.sync_copy(zbuf, rssh.at[pl.ds(s * ZB, ZB)])
    plsc.subcore_barrier()

    def chunk(g, _):
        pltpu.sync_copy(row3.at[w, g], ridx)
        pltpu.sync_copy(col3.at[w, g], cidx)
        cp1 = pltpu.async_copy(xn.at[ridx], arows, sem1)
        cp2 = pltpu.async_copy(xn.at[cidx], brows, sem2)
        cp1.wait()
        cp2.wait()

        lanes = lax.iota(jnp.int32, 16)

        def grp16(i, _):
            e16 = i * 16 + lanes

            def dstep(d, carry):
                acc, d16 = carry
                va = plsc.load_gather(arows, [e16, d16])
                vb = plsc.load_gather(brows, [e16, d16])
                return acc + va * vb, d16 + 1

            acc, _ = lax.fori_loop(
                0, D, dstep,
                (jnp.zeros((16,), jnp.float32), jnp.zeros((16,), jnp.int32)),
                unroll=16)
            attv[pl.ds(i * 16, 16)] = jnp.where(acc < THRESH, 0.0, acc)
            return 0
        lax.fori_loop(0, CH // 16, grp16, 0)

        pltpu.sync_copy(attv, att_out.at[w, g])
        pltpu.sync_copy(attv, rssh.at[ridx], add=True)
        return 0
    lax.fori_loop(0, NCH, chunk, 0)

    plsc.subcore_barrier()
    pltpu.sync_copy(rssh.at[pl.ds(s * ZB, ZB)], rs_out.at[c, pl.ds(s * ZB, ZB)])


@functools.partial(
    pl.kernel,
    out_type=jax.ShapeDtypeStruct((NC, NPAD), jnp.float32),
    mesh=_mesh,
    compiler_params=pltpu.CompilerParams(needs_layout_passes=False),
    scratch_types=(
        pltpu.VMEM((CH,), jnp.int32),
        pltpu.VMEM((CH,), jnp.int32),
        pltpu.VMEM((CH,), jnp.float32),
        pltpu.VMEM((CH,), jnp.float32),
        pltpu.VMEM((NPAD,), jnp.float32),
        pltpu.VMEM((ZB,), jnp.float32),
        pltpu.VMEM_SHARED((NPAD,), jnp.float32),
        pltpu.SemaphoreType.DMA,
    ),
)
def _sc_degree(att3, row3, col3, invrow, deg_out, ridx, cidx, attv, uv,
               irtab, zbuf, degsh, sem1):
    c = lax.axis_index("c")
    s = lax.axis_index("s")
    w = s * NC + c

    pltpu.sync_copy(invrow, irtab)
    _zero_vec(zbuf, ZB)
    pltpu.sync_copy(zbuf, degsh.at[pl.ds(s * ZB, ZB)])
    plsc.subcore_barrier()

    def chunk(g, _):
        pltpu.sync_copy(row3.at[w, g], ridx)
        pltpu.sync_copy(col3.at[w, g], cidx)
        pltpu.sync_copy(att3.at[w, g], attv)

        def grp(i, _):
            sl = pl.ds(i * 16, 16)
            r16 = ridx[sl]
            ir = plsc.load_gather(irtab, [r16])
            uv[sl] = attv[sl] * ir
            return 0
        lax.fori_loop(0, CH // 16, grp, 0)

        pltpu.sync_copy(uv, degsh.at[cidx], add=True)
        return 0
    lax.fori_loop(0, NCH, chunk, 0)

    plsc.subcore_barrier()
    pltpu.sync_copy(degsh.at[pl.ds(s * ZB, ZB)], deg_out.at[c, pl.ds(s * ZB, ZB)])


@functools.partial(
    pl.kernel,
    out_type=jax.ShapeDtypeStruct((NC, NPAD, D), jnp.float32),
    mesh=_mesh,
    compiler_params=pltpu.CompilerParams(needs_layout_passes=False),
    scratch_types=(
        pltpu.VMEM((CH,), jnp.int32),
        pltpu.VMEM((CH,), jnp.int32),
        pltpu.VMEM((CH,), jnp.float32),
        pltpu.VMEM((CH,), jnp.float32),
        pltpu.VMEM((NPAD,), jnp.float32),
        pltpu.VMEM((NPAD,), jnp.float32),
        pltpu.VMEM((CH, D), jnp.float32),
        pltpu.VMEM_SHARED((NPAD, D), jnp.float32),
        pltpu.SemaphoreType.DMA,
    ),
)
def _sc_aggregate(x, att3, row3, col3, dr, dv, acc_out, ridx, cidx, attv,
                  normv, drtab, dvtab, xr, accsh, sem1):
    c = lax.axis_index("c")
    s = lax.axis_index("s")
    w = s * NC + c

    pltpu.sync_copy(dr, drtab)
    pltpu.sync_copy(dv, dvtab)

    def zrow(e, _):
        for j in range(D // 16):
            xr[e, pl.ds(16 * j, 16)] = jnp.zeros((16,), jnp.float32)
        return 0
    lax.fori_loop(0, CH, zrow, 0)
    for k in range(ZB // CH):
        pltpu.sync_copy(xr, accsh.at[pl.ds(s * ZB + k * CH, CH)])
    plsc.subcore_barrier()

    def chunk(g, _):
        pltpu.sync_copy(row3.at[w, g], ridx)
        pltpu.sync_copy(col3.at[w, g], cidx)
        pltpu.sync_copy(att3.at[w, g], attv)
        pltpu.async_copy(x.at[ridx], xr, sem1).wait()

        def grp(i, _):
            sl = pl.ds(i * 16, 16)
            r16 = ridx[sl]
            c16 = cidx[sl]
            n16 = plsc.load_gather(drtab, [r16]) * attv[sl]
            n16 = n16 * plsc.load_gather(dvtab, [c16])
            normv[sl] = n16
            return 0
        lax.fori_loop(0, CH // 16, grp, 0)

        def scale(e, _):
            eidx = jnp.zeros((16,), jnp.int32) + e
            spl = plsc.load_gather(normv, [eidx])
            for j in range(D // 16):
                csl = pl.ds(16 * j, 16)
                xr[e, csl] = xr[e, csl] * spl
            return 0
        lax.fori_loop(0, CH, scale, 0)

        pltpu.sync_copy(xr, accsh.at[cidx], add=True)
        return 0
    lax.fori_loop(0, NCH, chunk, 0)

    plsc.subcore_barrier()
    pltpu.sync_copy(accsh.at[pl.ds(s * ZB, ZB)], acc_out.at[c, pl.ds(s * ZB, ZB)])


def _norm_body(x_ref, o_ref):
    xb = x_ref[...]
    ss = jnp.sum(xb * xb, axis=1, keepdims=True)
    o_ref[...] = xb * lax.rsqrt(jnp.maximum(ss, 1e-12))


def _normalize(x):
    return pl.pallas_call(
        _norm_body,
        grid=(10,),
        in_specs=[pl.BlockSpec((N // 10, D), lambda i: (i, 0))],
        out_specs=pl.BlockSpec((N // 10, D), lambda i: (i, 0)),
        out_shape=jax.ShapeDtypeStruct((N, D), jnp.float32),
    )(x)


def _invrow_body(p_ref, o_ref):
    rs = p_ref[0] + p_ref[1]
    o_ref[...] = 1.0 / jnp.maximum(rs, 1e-12)


def _vec_invrow(rsparts):
    return pl.pallas_call(
        _invrow_body,
        out_shape=jax.ShapeDtypeStruct((NPAD // D, D), jnp.float32),
    )(rsparts.reshape(NC, NPAD // D, D))


def _deg_body(p_ref, ir_ref, dv_ref, dr_ref, sw_ref):
    deg = p_ref[0] + p_ref[1] + 1.0
    dv = lax.rsqrt(deg)
    dv_ref[...] = dv
    dr_ref[...] = dv * ir_ref[...]
    sw_ref[...] = 1.0 / deg


def _vec_deg(degparts, invrow):
    shp = jax.ShapeDtypeStruct((NPAD // D, D), jnp.float32)
    return pl.pallas_call(
        _deg_body,
        out_shape=(shp, shp, shp),
    )(degparts.reshape(NC, NPAD // D, D), invrow)


def _mm_body(a0_ref, a1_ref, x_ref, sw_ref, w_ref, b_ref, o_ref, *, relu):
    a = a0_ref[...] + a1_ref[...] + x_ref[...] * sw_ref[...]
    h = jnp.dot(a, w_ref[...], preferred_element_type=jnp.float32) + b_ref[...]
    o_ref[...] = jnp.maximum(h, 0.0) if relu else h


def _mm(a0, a1, xpad, sw, w, b, relu):
    blk = NPAD // 10
    return pl.pallas_call(
        functools.partial(_mm_body, relu=relu),
        grid=(10,),
        in_specs=[
            pl.BlockSpec((blk, D), lambda i: (i, 0)),
            pl.BlockSpec((blk, D), lambda i: (i, 0)),
            pl.BlockSpec((blk, D), lambda i: (i, 0)),
            pl.BlockSpec((blk, 1), lambda i: (i, 0)),
            pl.BlockSpec((D, D), lambda i: (0, 0)),
            pl.BlockSpec((1, D), lambda i: (0, 0)),
        ],
        out_specs=pl.BlockSpec((blk, D), lambda i: (i, 0)),
        out_shape=jax.ShapeDtypeStruct((NPAD, D), jnp.float32),
    )(a0, a1, xpad, sw, w, b)


def _layer(x, xn, row3, col3, w_mat, b_vec, relu):
    att3, rsparts = _sc_attention(xn, row3, col3)
    invrow = _vec_invrow(rsparts)
    degparts = _sc_degree(att3, row3, col3, invrow.reshape(NPAD))
    dinv, dr, sw = _vec_deg(degparts, invrow)
    accparts = _sc_aggregate(x, att3, row3, col3, dr.reshape(NPAD), dinv.reshape(NPAD))
    xpad = jnp.pad(x, ((0, NPAD - N), (0, 0)))
    h = _mm(accparts[0], accparts[1], xpad, sw.reshape(NPAD, 1),
            w_mat, b_vec.reshape(1, D), relu)
    return h[:N]


def kernel(x, edge_index, W1, b1, W2, b2):
    row3 = edge_index[0].astype(jnp.int32).reshape(NW, NCH, CH)
    col3 = edge_index[1].astype(jnp.int32).reshape(NW, NCH, CH)
    xn = _normalize(x)
    h1 = _layer(x, xn, row3, col3, W1, b1, relu=True)
    out = _layer(h1, _normalize(h1), row3, col3, W2, b2, relu=False)
    return out


# speedup vs baseline: 10.2853x; 2.1286x over previous
import functools

import jax
import jax.numpy as jnp
from jax import lax
from jax.experimental import pallas as pl
from jax.experimental.pallas import tpu as pltpu
from jax.experimental.pallas import tpu_sc as plsc

N = 10000
E = 320000
D = 128
THRESH = 0.1
NC = 2
NS = 16
NW = NC * NS
EPW = E // NW
CH = 80
NCH = EPW // CH
NPAD = 10240
ZB = NPAD // NS

_mesh = plsc.VectorSubcoreMesh(core_axis_name="c", subcore_axis_name="s")


def _zero_vec(ref, nwords):
    def body(i, _):
        ref[pl.ds(i * 16, 16)] = jnp.zeros((16,), jnp.float32)
        return 0
    lax.fori_loop(0, nwords // 16, body, 0)


@functools.partial(
    pl.kernel,
    out_type=(
        jax.ShapeDtypeStruct((NW, NCH, CH), jnp.float32),
        jax.ShapeDtypeStruct((NC, NPAD), jnp.float32),
    ),
    mesh=_mesh,
    compiler_params=pltpu.CompilerParams(needs_layout_passes=False),
    scratch_types=(
        pltpu.VMEM((NCH, CH), jnp.int32),
        pltpu.VMEM((NCH, CH), jnp.int32),
        pltpu.VMEM((2, CH), jnp.float32),
        pltpu.VMEM((2, CH, D), jnp.float32),
        pltpu.VMEM((2, CH, D), jnp.float32),
        pltpu.VMEM((ZB,), jnp.float32),
        pltpu.VMEM_SHARED((NPAD,), jnp.float32),
        pltpu.SemaphoreType.DMA,
        pltpu.SemaphoreType.DMA,
        pltpu.SemaphoreType.DMA,
        pltpu.SemaphoreType.DMA,
        pltpu.SemaphoreType.DMA,
        pltpu.SemaphoreType.DMA,
    ),
)
def _sc_attention(xn, row3, col3, att_out, rs_out, rixs, cixs, attv,
                  arows, brows, zbuf, rssh, sa0, sb0, sa1, sb1, so0, so1):
    c = lax.axis_index("c")
    s = lax.axis_index("s")
    w = s * NC + c

    _zero_vec(zbuf, ZB)
    pltpu.sync_copy(zbuf, rssh.at[pl.ds(s * ZB, ZB)])
    plsc.subcore_barrier()

    pltpu.sync_copy(row3.at[w], rixs)
    pltpu.sync_copy(col3.at[w], cixs)

    lanes = lax.iota(jnp.int32, 16)
    sems = ((sa0, sb0, so0), (sa1, sb1, so1))

    def issue(g, b):
        pltpu.async_copy(xn.at[rixs.at[g]], arows.at[b], sems[b][0])
        pltpu.async_copy(xn.at[cixs.at[g]], brows.at[b], sems[b][1])

    issue(0, 0)

    def do_chunk(g, b):
        ar = arows.at[b]
        br = brows.at[b]
        ab = attv.at[b]
        pltpu.make_async_copy(xn.at[rixs.at[g]], ar, sems[b][0]).wait()
        pltpu.make_async_copy(xn.at[cixs.at[g]], br, sems[b][1]).wait()

        def grp16(i, _):
            v = jnp.zeros((16,), jnp.float32)
            for l in range(16):
                e = i * 16 + l
                acc = ar[e, pl.ds(0, 16)] * br[e, pl.ds(0, 16)]
                for j in range(1, 8):
                    acc = acc + ar[e, pl.ds(16 * j, 16)] * br[e, pl.ds(16 * j, 16)]
                v = jnp.where(lanes == l, jnp.sum(acc), v)
            ab[pl.ds(i * 16, 16)] = jnp.where(v < THRESH, 0.0, v)
            return 0
        lax.fori_loop(0, CH // 16, grp16, 0)

        pltpu.async_copy(ab, att_out.at[w, g], sems[b][2])
        pltpu.sync_copy(ab, rssh.at[rixs.at[g]], add=True)

    def body(g, _):
        for par in range(2):
            @pl.when(lax.rem(g, 2) == par)
            def _():
                @pl.when(g + 1 < NCH)
                def _():
                    issue(g + 1, 1 - par)
                @pl.when(g >= 2)
                def _():
                    pltpu.make_async_copy(
                        attv.at[par], att_out.at[w, g], sems[par][2]).wait()
                do_chunk(g, par)
        return 0
    lax.fori_loop(0, NCH, body, 0)
    pltpu.make_async_copy(attv.at[NCH % 2], att_out.at[w, 0],
                          sems[NCH % 2][2]).wait()
    pltpu.make_async_copy(attv.at[1 - NCH % 2], att_out.at[w, 0],
                          sems[1 - NCH % 2][2]).wait()

    plsc.subcore_barrier()
    pltpu.sync_copy(rssh.at[pl.ds(s * ZB, ZB)], rs_out.at[c, pl.ds(s * ZB, ZB)])


@functools.partial(
    pl.kernel,
    out_type=jax.ShapeDtypeStruct((NC, NPAD), jnp.float32),
    mesh=_mesh,
    compiler_params=pltpu.CompilerParams(needs_layout_passes=False),
    scratch_types=(
        pltpu.VMEM((CH,), jnp.int32),
        pltpu.VMEM((CH,), jnp.int32),
        pltpu.VMEM((CH,), jnp.float32),
        pltpu.VMEM((CH,), jnp.float32),
        pltpu.VMEM((NPAD,), jnp.float32),
        pltpu.VMEM((ZB,), jnp.float32),
        pltpu.VMEM_SHARED((NPAD,), jnp.float32),
        pltpu.SemaphoreType.DMA,
    ),
)
def _sc_degree(att3, row3, col3, invrow, deg_out, ridx, cidx, attv, uv,
               irtab, zbuf, degsh, sem1):
    c = lax.axis_index("c")
    s = lax.axis_index("s")
    w = s * NC + c

    pltpu.sync_copy(invrow, irtab)
    _zero_vec(zbuf, ZB)
    pltpu.sync_copy(zbuf, degsh.at[pl.ds(s * ZB, ZB)])
    plsc.subcore_barrier()

    def chunk(g, _):
        pltpu.sync_copy(row3.at[w, g], ridx)
        pltpu.sync_copy(col3.at[w, g], cidx)
        pltpu.sync_copy(att3.at[w, g], attv)

        def grp(i, _):
            sl = pl.ds(i * 16, 16)
            r16 = ridx[sl]
            ir = plsc.load_gather(irtab, [r16])
            uv[sl] = attv[sl] * ir
            return 0
        lax.fori_loop(0, CH // 16, grp, 0)

        pltpu.sync_copy(uv, degsh.at[cidx], add=True)
        return 0
    lax.fori_loop(0, NCH, chunk, 0)

    plsc.subcore_barrier()
    pltpu.sync_copy(degsh.at[pl.ds(s * ZB, ZB)], deg_out.at[c, pl.ds(s * ZB, ZB)])


@functools.partial(
    pl.kernel,
    out_type=jax.ShapeDtypeStruct((NC, NPAD, D), jnp.float32),
    mesh=_mesh,
    compiler_params=pltpu.CompilerParams(needs_layout_passes=False),
    scratch_types=(
        pltpu.VMEM((CH,), jnp.int32),
        pltpu.VMEM((CH,), jnp.int32),
        pltpu.VMEM((CH,), jnp.float32),
        pltpu.VMEM((CH,), jnp.float32),
        pltpu.VMEM((NPAD,), jnp.float32),
        pltpu.VMEM((NPAD,), jnp.float32),
        pltpu.VMEM((CH, D), jnp.float32),
        pltpu.VMEM_SHARED((NPAD, D), jnp.float32),
        pltpu.SemaphoreType.DMA,
    ),
)
def _sc_aggregate(x, att3, row3, col3, dr, dv, acc_out, ridx, cidx, attv,
                  normv, drtab, dvtab, xr, accsh, sem1):
    c = lax.axis_index("c")
    s = lax.axis_index("s")
    w = s * NC + c

    pltpu.sync_copy(dr, drtab)
    pltpu.sync_copy(dv, dvtab)

    def zrow(e, _):
        for j in range(D // 16):
            xr[e, pl.ds(16 * j, 16)] = jnp.zeros((16,), jnp.float32)
        return 0
    lax.fori_loop(0, CH, zrow, 0)
    for k in range(ZB // CH):
        pltpu.sync_copy(xr, accsh.at[pl.ds(s * ZB + k * CH, CH)])
    plsc.subcore_barrier()

    def chunk(g, _):
        pltpu.sync_copy(row3.at[w, g], ridx)
        pltpu.sync_copy(col3.at[w, g], cidx)
        pltpu.sync_copy(att3.at[w, g], attv)
        pltpu.async_copy(x.at[ridx], xr, sem1).wait()

        def grp(i, _):
            sl = pl.ds(i * 16, 16)
            r16 = ridx[sl]
            c16 = cidx[sl]
            n16 = plsc.load_gather(drtab, [r16]) * attv[sl]
            n16 = n16 * plsc.load_gather(dvtab, [c16])
            normv[sl] = n16
            return 0
        lax.fori_loop(0, CH // 16, grp, 0)

        def scale(e, _):
            eidx = jnp.zeros((16,), jnp.int32) + e
            spl = plsc.load_gather(normv, [eidx])
            for j in range(D // 16):
                csl = pl.ds(16 * j, 16)
                xr[e, csl] = xr[e, csl] * spl
            return 0
        lax.fori_loop(0, CH, scale, 0)

        pltpu.sync_copy(xr, accsh.at[cidx], add=True)
        return 0
    lax.fori_loop(0, NCH, chunk, 0)

    plsc.subcore_barrier()
    pltpu.sync_copy(accsh.at[pl.ds(s * ZB, ZB)], acc_out.at[c, pl.ds(s * ZB, ZB)])


def _norm_body(x_ref, o_ref):
    xb = x_ref[...]
    ss = jnp.sum(xb * xb, axis=1, keepdims=True)
    o_ref[...] = xb * lax.rsqrt(jnp.maximum(ss, 1e-12))


def _normalize(x):
    return pl.pallas_call(
        _norm_body,
        grid=(10,),
        in_specs=[pl.BlockSpec((N // 10, D), lambda i: (i, 0))],
        out_specs=pl.BlockSpec((N // 10, D), lambda i: (i, 0)),
        out_shape=jax.ShapeDtypeStruct((N, D), jnp.float32),
    )(x)


def _invrow_body(p_ref, o_ref):
    rs = p_ref[0] + p_ref[1]
    o_ref[...] = 1.0 / jnp.maximum(rs, 1e-12)


def _vec_invrow(rsparts):
    return pl.pallas_call(
        _invrow_body,
        out_shape=jax.ShapeDtypeStruct((NPAD // D, D), jnp.float32),
    )(rsparts.reshape(NC, NPAD // D, D))


def _deg_body(p_ref, ir_ref, dv_ref, dr_ref, sw_ref):
    deg = p_ref[0] + p_ref[1] + 1.0
    dv = lax.rsqrt(deg)
    dv_ref[...] = dv
    dr_ref[...] = dv * ir_ref[...]
    sw_ref[...] = 1.0 / deg


def _vec_deg(degparts, invrow):
    shp = jax.ShapeDtypeStruct((NPAD // D, D), jnp.float32)
    return pl.pallas_call(
        _deg_body,
        out_shape=(shp, shp, shp),
    )(degparts.reshape(NC, NPAD // D, D), invrow)


def _mm_body(a0_ref, a1_ref, x_ref, sw_ref, w_ref, b_ref, o_ref, *, relu):
    a = a0_ref[...] + a1_ref[...] + x_ref[...] * sw_ref[...]
    h = jnp.dot(a, w_ref[...], preferred_element_type=jnp.float32) + b_ref[...]
    o_ref[...] = jnp.maximum(h, 0.0) if relu else h


def _mm(a0, a1, xpad, sw, w, b, relu):
    blk = NPAD // 10
    return pl.pallas_call(
        functools.partial(_mm_body, relu=relu),
        grid=(10,),
        in_specs=[
            pl.BlockSpec((blk, D), lambda i: (i, 0)),
            pl.BlockSpec((blk, D), lambda i: (i, 0)),
            pl.BlockSpec((blk, D), lambda i: (i, 0)),
            pl.BlockSpec((blk, 1), lambda i: (i, 0)),
            pl.BlockSpec((D, D), lambda i: (0, 0)),
            pl.BlockSpec((1, D), lambda i: (0, 0)),
        ],
        out_specs=pl.BlockSpec((blk, D), lambda i: (i, 0)),
        out_shape=jax.ShapeDtypeStruct((NPAD, D), jnp.float32),
    )(a0, a1, xpad, sw, w, b)


def _layer(x, xn, row3, col3, w_mat, b_vec, relu):
    att3, rsparts = _sc_attention(xn, row3, col3)
    invrow = _vec_invrow(rsparts)
    degparts = _sc_degree(att3, row3, col3, invrow.reshape(NPAD))
    dinv, dr, sw = _vec_deg(degparts, invrow)
    accparts = _sc_aggregate(x, att3, row3, col3, dr.reshape(NPAD), dinv.reshape(NPAD))
    xpad = jnp.pad(x, ((0, NPAD - N), (0, 0)))
    h = _mm(accparts[0], accparts[1], xpad, sw.reshape(NPAD, 1),
            w_mat, b_vec.reshape(1, D), relu)
    return h[:N]


def kernel(x, edge_index, W1, b1, W2, b2):
    row3 = edge_index[0].astype(jnp.int32).reshape(NW, NCH, CH)
    col3 = edge_index[1].astype(jnp.int32).reshape(NW, NCH, CH)
    xn = _normalize(x)
    h1 = _layer(x, xn, row3, col3, W1, b1, relu=True)
    out = _layer(h1, _normalize(h1), row3, col3, W2, b2, relu=False)
    return out


# speedup vs baseline: 18.3396x; 1.7831x over previous
import functools

import jax
import jax.numpy as jnp
from jax import lax
from jax.experimental import pallas as pl
from jax.experimental.pallas import tpu as pltpu
from jax.experimental.pallas import tpu_sc as plsc

N = 10000
E = 320000
D = 128
THRESH = 0.1
NC = 2
NS = 16
NW = NC * NS
EPW = E // NW
CH = 80
NCH = EPW // CH
NPAD = 10240
ZB = NPAD // NS

_mesh = plsc.VectorSubcoreMesh(core_axis_name="c", subcore_axis_name="s")


def _zero_vec(ref, nwords):
    def body(i, _):
        ref[pl.ds(i * 16, 16)] = jnp.zeros((16,), jnp.float32)
        return 0
    lax.fori_loop(0, nwords // 16, body, 0)


@functools.partial(
    pl.kernel,
    out_type=(
        jax.ShapeDtypeStruct((NW, NCH, CH), jnp.float32),
        jax.ShapeDtypeStruct((NC, NPAD), jnp.float32),
    ),
    mesh=_mesh,
    compiler_params=pltpu.CompilerParams(needs_layout_passes=False),
    scratch_types=(
        pltpu.VMEM((NCH, CH), jnp.int32),
        pltpu.VMEM((NCH, CH), jnp.int32),
        pltpu.VMEM((2, CH), jnp.float32),
        pltpu.VMEM((2, CH, D), jnp.float32),
        pltpu.VMEM((2, CH, D), jnp.float32),
        pltpu.VMEM((ZB,), jnp.float32),
        pltpu.VMEM_SHARED((NPAD,), jnp.float32),
        pltpu.SemaphoreType.DMA,
        pltpu.SemaphoreType.DMA,
        pltpu.SemaphoreType.DMA,
        pltpu.SemaphoreType.DMA,
        pltpu.SemaphoreType.DMA,
        pltpu.SemaphoreType.DMA,
        pltpu.SemaphoreType.DMA,
        pltpu.SemaphoreType.DMA,
    ),
)
def _sc_attention(xn, row3, col3, att_out, rs_out, rixs, cixs, attv,
                  arows, brows, zbuf, rssh, sa0, sb0, sa1, sb1, so0, so1,
                  ss0, ss1):
    c = lax.axis_index("c")
    s = lax.axis_index("s")
    w = s * NC + c

    _zero_vec(zbuf, ZB)
    pltpu.sync_copy(zbuf, rssh.at[pl.ds(s * ZB, ZB)])
    plsc.subcore_barrier()

    pltpu.sync_copy(row3.at[w], rixs)
    pltpu.sync_copy(col3.at[w], cixs)

    lanes = lax.iota(jnp.int32, 16)
    sems = ((sa0, sb0, so0, ss0), (sa1, sb1, so1, ss1))

    def issue(g, b):
        pltpu.async_copy(xn.at[rixs.at[g]], arows.at[b], sems[b][0])
        pltpu.async_copy(xn.at[cixs.at[g]], brows.at[b], sems[b][1])

    issue(0, 0)

    def do_chunk(g, b):
        ar = arows.at[b]
        br = brows.at[b]
        ab = attv.at[b]
        pltpu.make_async_copy(xn.at[rixs.at[g]], ar, sems[b][0]).wait()
        pltpu.make_async_copy(xn.at[cixs.at[g]], br, sems[b][1]).wait()

        def grp16(i, _):
            v = jnp.zeros((16,), jnp.float32)
            for l in range(16):
                e = i * 16 + l
                acc = ar[e, pl.ds(0, 16)] * br[e, pl.ds(0, 16)]
                for j in range(1, 8):
                    acc = acc + ar[e, pl.ds(16 * j, 16)] * br[e, pl.ds(16 * j, 16)]
                v = jnp.where(lanes == l, jnp.sum(acc), v)
            ab[pl.ds(i * 16, 16)] = jnp.where(v < THRESH, 0.0, v)
            return 0
        lax.fori_loop(0, CH // 16, grp16, 0)

        pltpu.async_copy(ab, att_out.at[w, g], sems[b][2])
        pltpu.async_copy(ab, rssh.at[rixs.at[g]], sems[b][3], add=True)

    def body(g, _):
        for par in range(2):
            @pl.when(lax.rem(g, 2) == par)
            def _():
                @pl.when(g + 1 < NCH)
                def _():
                    issue(g + 1, 1 - par)
                @pl.when(g >= 2)
                def _():
                    pltpu.make_async_copy(
                        attv.at[par], att_out.at[w, g], sems[par][2]).wait()
                    pltpu.make_async_copy(
                        attv.at[par], rssh.at[rixs.at[g]], sems[par][3]).wait()
                do_chunk(g, par)
        return 0
    lax.fori_loop(0, NCH, body, 0)
    for par in range(2):
        pltpu.make_async_copy(attv.at[par], att_out.at[w, 0],
                              sems[par][2]).wait()
        pltpu.make_async_copy(attv.at[par], rssh.at[rixs.at[0]],
                              sems[par][3]).wait()

    plsc.subcore_barrier()
    pltpu.sync_copy(rssh.at[pl.ds(s * ZB, ZB)], rs_out.at[c, pl.ds(s * ZB, ZB)])


@functools.partial(
    pl.kernel,
    out_type=jax.ShapeDtypeStruct((NC, NPAD), jnp.float32),
    mesh=_mesh,
    compiler_params=pltpu.CompilerParams(needs_layout_passes=False),
    scratch_types=(
        pltpu.VMEM((NCH, CH), jnp.int32),
        pltpu.VMEM((NCH, CH), jnp.int32),
        pltpu.VMEM((NCH, CH), jnp.float32),
        pltpu.VMEM((2, CH), jnp.float32),
        pltpu.VMEM((NPAD,), jnp.float32),
        pltpu.VMEM((ZB,), jnp.float32),
        pltpu.VMEM_SHARED((NPAD,), jnp.float32),
        pltpu.SemaphoreType.DMA,
        pltpu.SemaphoreType.DMA,
    ),
)
def _sc_degree(att3, row3, col3, invrow, deg_out, rixs, cixs, atts, uv,
               irtab, zbuf, degsh, su0, su1):
    c = lax.axis_index("c")
    s = lax.axis_index("s")
    w = s * NC + c

    pltpu.sync_copy(invrow, irtab)
    _zero_vec(zbuf, ZB)
    pltpu.sync_copy(zbuf, degsh.at[pl.ds(s * ZB, ZB)])
    pltpu.sync_copy(row3.at[w], rixs)
    pltpu.sync_copy(col3.at[w], cixs)
    pltpu.sync_copy(att3.at[w], atts)
    plsc.subcore_barrier()

    sems = (su0, su1)

    def chunk(g, _):
        for par in range(2):
            @pl.when(lax.rem(g, 2) == par)
            def _():
                ub = uv.at[par]
                @pl.when(g >= 2)
                def _():
                    pltpu.make_async_copy(
                        ub, degsh.at[cixs.at[g]], sems[par]).wait()

                def grp(i, _):
                    sl = pl.ds(i * 16, 16)
                    r16 = rixs[g, sl]
                    ir = plsc.load_gather(irtab, [r16])
                    ub[sl] = atts[g, sl] * ir
                    return 0
                lax.fori_loop(0, CH // 16, grp, 0)

                pltpu.async_copy(ub, degsh.at[cixs.at[g]], sems[par], add=True)
        return 0
    lax.fori_loop(0, NCH, chunk, 0)
    for par in range(2):
        pltpu.make_async_copy(uv.at[par], degsh.at[cixs.at[0]],
                              sems[par]).wait()

    plsc.subcore_barrier()
    pltpu.sync_copy(degsh.at[pl.ds(s * ZB, ZB)], deg_out.at[c, pl.ds(s * ZB, ZB)])


@functools.partial(
    pl.kernel,
    out_type=jax.ShapeDtypeStruct((NC, NPAD, D), jnp.float32),
    mesh=_mesh,
    compiler_params=pltpu.CompilerParams(needs_layout_passes=False),
    scratch_types=(
        pltpu.VMEM((3, CH), jnp.int32),
        pltpu.VMEM((3, CH), jnp.int32),
        pltpu.VMEM((3, CH), jnp.float32),
        pltpu.VMEM((CH,), jnp.float32),
        pltpu.VMEM((NPAD,), jnp.float32),
        pltpu.VMEM((3, CH, D), jnp.float32),
        pltpu.VMEM_SHARED((NPAD, D), jnp.float32),
        pltpu.SemaphoreType.DMA,
        pltpu.SemaphoreType.DMA,
        pltpu.SemaphoreType.DMA,
        pltpu.SemaphoreType.DMA,
        pltpu.SemaphoreType.DMA,
        pltpu.SemaphoreType.DMA,
        pltpu.SemaphoreType.DMA,
        pltpu.SemaphoreType.DMA,
        pltpu.SemaphoreType.DMA,
    ),
)
def _sc_aggregate(x, att3, row3, col3, dr, acc_out, rixs, cixs, atts,
                  normv, drtab, xr, accsh,
                  sg0, sg1, sg2, ss0, ss1, ss2, si0, si1, si2):
    c = lax.axis_index("c")
    s = lax.axis_index("s")
    w = s * NC + c

    pltpu.sync_copy(dr, drtab)

    def zrow(e, _):
        for j in range(D // 16):
            xr[0, e, pl.ds(16 * j, 16)] = jnp.zeros((16,), jnp.float32)
        return 0
    lax.fori_loop(0, CH, zrow, 0)
    for k in range(ZB // CH):
        pltpu.sync_copy(xr.at[0], accsh.at[pl.ds(s * ZB + k * CH, CH)])
    plsc.subcore_barrier()

    sg = (sg0, sg1, sg2)
    ss = (ss0, ss1, ss2)
    si = (si0, si1, si2)

    def issue_idx(g, b):
        pltpu.async_copy(row3.at[w, g], rixs.at[b], si[b])
        pltpu.async_copy(col3.at[w, g], cixs.at[b], si[b])
        pltpu.async_copy(att3.at[w, g], atts.at[b], si[b])

    def wait_idx(g, b):
        pltpu.make_async_copy(row3.at[w, g], rixs.at[b], si[b]).wait()
        pltpu.make_async_copy(col3.at[w, g], cixs.at[b], si[b]).wait()
        pltpu.make_async_copy(att3.at[w, g], atts.at[b], si[b]).wait()

    issue_idx(0, 0)
    issue_idx(1, 1)
    wait_idx(0, 0)
    pltpu.async_copy(x.at[rixs.at[0]], xr.at[0], sg[0])

    def chunk(g, _):
        for par in range(3):
            @pl.when(lax.rem(g, 3) == par)
            def _():
                nb = (par + 1) % 3
                @pl.when(g + 1 < NCH)
                def _():
                    @pl.when(g >= 2)
                    def _():
                        pltpu.make_async_copy(
                            xr.at[nb], accsh.at[cixs.at[nb]], ss[nb]).wait()
                    wait_idx(g + 1, nb)
                    pltpu.async_copy(x.at[rixs.at[nb]], xr.at[nb], sg[nb])

                xb = xr.at[par]
                pltpu.make_async_copy(x.at[rixs.at[par]], xb, sg[par]).wait()

                def grp(i, _):
                    sl = pl.ds(i * 16, 16)
                    r16 = rixs[par, sl]
                    n16 = plsc.load_gather(drtab, [r16]) * atts[par, sl]
                    normv[sl] = n16
                    return 0
                lax.fori_loop(0, CH // 16, grp, 0)

                def scale(e, _):
                    eidx = jnp.zeros((16,), jnp.int32) + e
                    spl = plsc.load_gather(normv, [eidx])
                    for j in range(D // 16):
                        csl = pl.ds(16 * j, 16)
                        xb[e, csl] = xb[e, csl] * spl
                    return 0
                lax.fori_loop(0, CH, scale, 0)

                pltpu.async_copy(xb, accsh.at[cixs.at[par]], ss[par], add=True)

                @pl.when(g + 2 < NCH)
                def _():
                    issue_idx(g + 2, (par + 2) % 3)
        return 0
    lax.fori_loop(0, NCH, chunk, 0)
    for par in range(3):
        pltpu.make_async_copy(xr.at[par], accsh.at[cixs.at[par]],
                              ss[par]).wait()

    plsc.subcore_barrier()
    pltpu.sync_copy(accsh.at[pl.ds(s * ZB, ZB)], acc_out.at[c, pl.ds(s * ZB, ZB)])


def _norm_body(x_ref, o_ref):
    xb = x_ref[...]
    ss = jnp.sum(xb * xb, axis=1, keepdims=True)
    o_ref[...] = xb * lax.rsqrt(jnp.maximum(ss, 1e-12))


def _normalize(x):
    return pl.pallas_call(
        _norm_body,
        grid=(10,),
        in_specs=[pl.BlockSpec((N // 10, D), lambda i: (i, 0))],
        out_specs=pl.BlockSpec((N // 10, D), lambda i: (i, 0)),
        out_shape=jax.ShapeDtypeStruct((N, D), jnp.float32),
    )(x)


def _invrow_body(p_ref, o_ref):
    rs = p_ref[0] + p_ref[1]
    o_ref[...] = 1.0 / jnp.maximum(rs, 1e-12)


def _vec_invrow(rsparts):
    return pl.pallas_call(
        _invrow_body,
        out_shape=jax.ShapeDtypeStruct((NPAD // D, D), jnp.float32),
    )(rsparts.reshape(NC, NPAD // D, D))


def _deg_body(p_ref, ir_ref, dv_ref, dr_ref):
    deg = p_ref[0] + p_ref[1] + 1.0
    dv = lax.rsqrt(deg)
    dv_ref[...] = dv
    dr_ref[...] = dv * ir_ref[...]


def _vec_deg(degparts, invrow):
    shp = jax.ShapeDtypeStruct((NPAD // D, D), jnp.float32)
    return pl.pallas_call(
        _deg_body,
        out_shape=(shp, shp),
    )(degparts.reshape(NC, NPAD // D, D), invrow)


def _mm_body(a0_ref, a1_ref, x_ref, dv_ref, w_ref, b_ref, o_ref, *, relu):
    dv = dv_ref[...]
    a = (a0_ref[...] + a1_ref[...] + x_ref[...] * dv) * dv
    h = jnp.dot(a, w_ref[...], preferred_element_type=jnp.float32) + b_ref[...]
    o_ref[...] = jnp.maximum(h, 0.0) if relu else h


def _mm(a0, a1, xpad, sw, w, b, relu):
    blk = NPAD // 10
    return pl.pallas_call(
        functools.partial(_mm_body, relu=relu),
        grid=(10,),
        in_specs=[
            pl.BlockSpec((blk, D), lambda i: (i, 0)),
            pl.BlockSpec((blk, D), lambda i: (i, 0)),
            pl.BlockSpec((blk, D), lambda i: (i, 0)),
            pl.BlockSpec((blk, 1), lambda i: (i, 0)),
            pl.BlockSpec((D, D), lambda i: (0, 0)),
            pl.BlockSpec((1, D), lambda i: (0, 0)),
        ],
        out_specs=pl.BlockSpec((blk, D), lambda i: (i, 0)),
        out_shape=jax.ShapeDtypeStruct((NPAD, D), jnp.float32),
    )(a0, a1, xpad, sw, w, b)


def _layer(x, xn, row3, col3, w_mat, b_vec, relu):
    att3, rsparts = _sc_attention(xn, row3, col3)
    invrow = _vec_invrow(rsparts)
    degparts = _sc_degree(att3, row3, col3, invrow.reshape(NPAD))
    dinv, dr = _vec_deg(degparts, invrow)
    accparts = _sc_aggregate(x, att3, row3, col3, dr.reshape(NPAD))
    xpad = jnp.pad(x, ((0, NPAD - N), (0, 0)))
    h = _mm(accparts[0], accparts[1], xpad, dinv.reshape(NPAD, 1),
            w_mat, b_vec.reshape(1, D), relu)
    return h[:N]


def kernel(x, edge_index, W1, b1, W2, b2):
    row3 = edge_index[0].astype(jnp.int32).reshape(NW, NCH, CH)
    col3 = edge_index[1].astype(jnp.int32).reshape(NW, NCH, CH)
    xn = _normalize(x)
    h1 = _layer(x, xn, row3, col3, W1, b1, relu=True)
    out = _layer(h1, _normalize(h1), row3, col3, W2, b2, relu=False)
    return out


# speedup vs baseline: 22.1601x; 1.2083x over previous
import functools

import jax
import jax.numpy as jnp
from jax import lax
from jax.experimental import pallas as pl
from jax.experimental.pallas import tpu as pltpu
from jax.experimental.pallas import tpu_sc as plsc

N = 10000
E = 320000
D = 128
THRESH = 0.1
NC = 2
NS = 16
NW = NC * NS
EPW = E // NW
CH = 80
NCH = EPW // CH
NPAD = 10240
ZB = NPAD // NS

_mesh = plsc.VectorSubcoreMesh(core_axis_name="c", subcore_axis_name="s")


def _zero_vec(ref, nwords):
    def body(i, _):
        ref[pl.ds(i * 16, 16)] = jnp.zeros((16,), jnp.float32)
        return 0
    lax.fori_loop(0, nwords // 16, body, 0)


@functools.partial(
    pl.kernel,
    out_type=(
        jax.ShapeDtypeStruct((NW, NCH, CH), jnp.float32),
        jax.ShapeDtypeStruct((NC, NPAD), jnp.float32),
    ),
    mesh=_mesh,
    compiler_params=pltpu.CompilerParams(needs_layout_passes=False),
    scratch_types=(
        pltpu.VMEM((NCH, CH), jnp.int32),
        pltpu.VMEM((NCH, CH), jnp.int32),
        pltpu.VMEM((2, CH), jnp.float32),
        pltpu.VMEM((2, CH, D), jnp.float32),
        pltpu.VMEM((2, CH, D), jnp.float32),
        pltpu.VMEM((16, 17), jnp.float32),
        pltpu.VMEM((ZB,), jnp.float32),
        pltpu.VMEM_SHARED((NPAD,), jnp.float32),
        pltpu.SemaphoreType.DMA,
        pltpu.SemaphoreType.DMA,
        pltpu.SemaphoreType.DMA,
        pltpu.SemaphoreType.DMA,
        pltpu.SemaphoreType.DMA,
        pltpu.SemaphoreType.DMA,
        pltpu.SemaphoreType.DMA,
        pltpu.SemaphoreType.DMA,
    ),
)
def _sc_attention(xn, row3, col3, att_out, rs_out, rixs, cixs, attv,
                  arows, brows, tbuf, zbuf, rssh, sa0, sb0, sa1, sb1, so0, so1,
                  ss0, ss1):
    c = lax.axis_index("c")
    s = lax.axis_index("s")
    w = s * NC + c

    _zero_vec(zbuf, ZB)
    pltpu.sync_copy(zbuf, rssh.at[pl.ds(s * ZB, ZB)])
    plsc.subcore_barrier()

    pltpu.sync_copy(row3.at[w], rixs)
    pltpu.sync_copy(col3.at[w], cixs)

    lanes = lax.iota(jnp.int32, 16)
    sems = ((sa0, sb0, so0, ss0), (sa1, sb1, so1, ss1))

    def issue(g, b):
        pltpu.async_copy(xn.at[rixs.at[g]], arows.at[b], sems[b][0])
        pltpu.async_copy(xn.at[cixs.at[g]], brows.at[b], sems[b][1])

    issue(0, 0)

    def do_chunk(g, b):
        ar = arows.at[b]
        br = brows.at[b]
        ab = attv.at[b]
        pltpu.make_async_copy(xn.at[rixs.at[g]], ar, sems[b][0]).wait()
        pltpu.make_async_copy(xn.at[cixs.at[g]], br, sems[b][1]).wait()

        def grp16(i, _):
            for l in range(16):
                e = i * 16 + l
                acc = ar[e, pl.ds(0, 16)] * br[e, pl.ds(0, 16)]
                for j in range(1, 8):
                    acc = acc + ar[e, pl.ds(16 * j, 16)] * br[e, pl.ds(16 * j, 16)]
                tbuf[l, pl.ds(0, 16)] = acc
            cols = [plsc.load_gather(tbuf, [lanes, jnp.full((16,), j, jnp.int32)])
                    for j in range(16)]
            while len(cols) > 1:
                cols = [cols[k] + cols[k + 1] for k in range(0, len(cols), 2)]
            v = cols[0]
            ab[pl.ds(i * 16, 16)] = jnp.where(v < THRESH, 0.0, v)
            return 0
        lax.fori_loop(0, CH // 16, grp16, 0)

        pltpu.async_copy(ab, att_out.at[w, g], sems[b][2])
        pltpu.async_copy(ab, rssh.at[rixs.at[g]], sems[b][3], add=True)

    def body(g, _):
        for par in range(2):
            @pl.when(lax.rem(g, 2) == par)
            def _():
                @pl.when(g + 1 < NCH)
                def _():
                    issue(g + 1, 1 - par)
                @pl.when(g >= 2)
                def _():
                    pltpu.make_async_copy(
                        attv.at[par], att_out.at[w, g], sems[par][2]).wait()
                    pltpu.make_async_copy(
                        attv.at[par], rssh.at[rixs.at[g]], sems[par][3]).wait()
                do_chunk(g, par)
        return 0
    lax.fori_loop(0, NCH, body, 0)
    for par in range(2):
        pltpu.make_async_copy(attv.at[par], att_out.at[w, 0],
                              sems[par][2]).wait()
        pltpu.make_async_copy(attv.at[par], rssh.at[rixs.at[0]],
                              sems[par][3]).wait()

    plsc.subcore_barrier()
    pltpu.sync_copy(rssh.at[pl.ds(s * ZB, ZB)], rs_out.at[c, pl.ds(s * ZB, ZB)])


@functools.partial(
    pl.kernel,
    out_type=jax.ShapeDtypeStruct((NC, NPAD), jnp.float32),
    mesh=_mesh,
    compiler_params=pltpu.CompilerParams(needs_layout_passes=False),
    scratch_types=(
        pltpu.VMEM((NCH, CH), jnp.int32),
        pltpu.VMEM((NCH, CH), jnp.int32),
        pltpu.VMEM((NCH, CH), jnp.float32),
        pltpu.VMEM((2, CH), jnp.float32),
        pltpu.VMEM((NPAD,), jnp.float32),
        pltpu.VMEM((ZB,), jnp.float32),
        pltpu.VMEM_SHARED((NPAD,), jnp.float32),
        pltpu.SemaphoreType.DMA,
        pltpu.SemaphoreType.DMA,
    ),
)
def _sc_degree(att3, row3, col3, invrow, deg_out, rixs, cixs, atts, uv,
               irtab, zbuf, degsh, su0, su1):
    c = lax.axis_index("c")
    s = lax.axis_index("s")
    w = s * NC + c

    pltpu.sync_copy(invrow, irtab)
    _zero_vec(zbuf, ZB)
    pltpu.sync_copy(zbuf, degsh.at[pl.ds(s * ZB, ZB)])
    pltpu.sync_copy(row3.at[w], rixs)
    pltpu.sync_copy(col3.at[w], cixs)
    pltpu.sync_copy(att3.at[w], atts)
    plsc.subcore_barrier()

    sems = (su0, su1)

    def chunk(g, _):
        for par in range(2):
            @pl.when(lax.rem(g, 2) == par)
            def _():
                ub = uv.at[par]
                @pl.when(g >= 2)
                def _():
                    pltpu.make_async_copy(
                        ub, degsh.at[cixs.at[g]], sems[par]).wait()

                def grp(i, _):
                    sl = pl.ds(i * 16, 16)
                    r16 = rixs[g, sl]
                    ir = plsc.load_gather(irtab, [r16])
                    ub[sl] = atts[g, sl] * ir
                    return 0
                lax.fori_loop(0, CH // 16, grp, 0)

                pltpu.async_copy(ub, degsh.at[cixs.at[g]], sems[par], add=True)
        return 0
    lax.fori_loop(0, NCH, chunk, 0)
    for par in range(2):
        pltpu.make_async_copy(uv.at[par], degsh.at[cixs.at[0]],
                              sems[par]).wait()

    plsc.subcore_barrier()
    pltpu.sync_copy(degsh.at[pl.ds(s * ZB, ZB)], deg_out.at[c, pl.ds(s * ZB, ZB)])


@functools.partial(
    pl.kernel,
    out_type=jax.ShapeDtypeStruct((NC, NPAD, D), jnp.float32),
    mesh=_mesh,
    compiler_params=pltpu.CompilerParams(needs_layout_passes=False),
    scratch_types=(
        pltpu.VMEM((3, CH), jnp.int32),
        pltpu.VMEM((3, CH), jnp.int32),
        pltpu.VMEM((3, CH), jnp.float32),
        pltpu.VMEM((CH,), jnp.float32),
        pltpu.VMEM((NPAD,), jnp.float32),
        pltpu.VMEM((3, CH, D), jnp.float32),
        pltpu.VMEM_SHARED((NPAD, D), jnp.float32),
        pltpu.SemaphoreType.DMA,
        pltpu.SemaphoreType.DMA,
        pltpu.SemaphoreType.DMA,
        pltpu.SemaphoreType.DMA,
        pltpu.SemaphoreType.DMA,
        pltpu.SemaphoreType.DMA,
        pltpu.SemaphoreType.DMA,
        pltpu.SemaphoreType.DMA,
        pltpu.SemaphoreType.DMA,
    ),
)
def _sc_aggregate(x, att3, row3, col3, dr, acc_out, rixs, cixs, atts,
                  normv, drtab, xr, accsh,
                  sg0, sg1, sg2, ss0, ss1, ss2, si0, si1, si2):
    c = lax.axis_index("c")
    s = lax.axis_index("s")
    w = s * NC + c

    pltpu.sync_copy(dr, drtab)

    def zrow(e, _):
        for j in range(D // 16):
            xr[0, e, pl.ds(16 * j, 16)] = jnp.zeros((16,), jnp.float32)
        return 0
    lax.fori_loop(0, CH, zrow, 0)
    for k in range(ZB // CH):
        pltpu.sync_copy(xr.at[0], accsh.at[pl.ds(s * ZB + k * CH, CH)])
    plsc.subcore_barrier()

    sg = (sg0, sg1, sg2)
    ss = (ss0, ss1, ss2)
    si = (si0, si1, si2)

    def issue_idx(g, b):
        pltpu.async_copy(row3.at[w, g], rixs.at[b], si[b])
        pltpu.async_copy(col3.at[w, g], cixs.at[b], si[b])
        pltpu.async_copy(att3.at[w, g], atts.at[b], si[b])

    def wait_idx(g, b):
        pltpu.make_async_copy(row3.at[w, g], rixs.at[b], si[b]).wait()
        pltpu.make_async_copy(col3.at[w, g], cixs.at[b], si[b]).wait()
        pltpu.make_async_copy(att3.at[w, g], atts.at[b], si[b]).wait()

    issue_idx(0, 0)
    issue_idx(1, 1)
    wait_idx(0, 0)
    pltpu.async_copy(x.at[rixs.at[0]], xr.at[0], sg[0])

    def chunk(g, _):
        for par in range(3):
            @pl.when(lax.rem(g, 3) == par)
            def _():
                nb = (par + 1) % 3
                @pl.when(g + 1 < NCH)
                def _():
                    @pl.when(g >= 2)
                    def _():
                        pltpu.make_async_copy(
                            xr.at[nb], accsh.at[cixs.at[nb]], ss[nb]).wait()
                    wait_idx(g + 1, nb)
                    pltpu.async_copy(x.at[rixs.at[nb]], xr.at[nb], sg[nb])

                xb = xr.at[par]
                pltpu.make_async_copy(x.at[rixs.at[par]], xb, sg[par]).wait()

                def grp(i, _):
                    sl = pl.ds(i * 16, 16)
                    r16 = rixs[par, sl]
                    n16 = plsc.load_gather(drtab, [r16]) * atts[par, sl]
                    normv[sl] = n16
                    return 0
                lax.fori_loop(0, CH // 16, grp, 0)

                def scale(e, _):
                    eidx = jnp.zeros((16,), jnp.int32) + e
                    spl = plsc.load_gather(normv, [eidx])
                    for j in range(D // 16):
                        csl = pl.ds(16 * j, 16)
                        xb[e, csl] = xb[e, csl] * spl
                    return 0
                lax.fori_loop(0, CH, scale, 0)

                pltpu.async_copy(xb, accsh.at[cixs.at[par]], ss[par], add=True)

                @pl.when(g + 2 < NCH)
                def _():
                    issue_idx(g + 2, (par + 2) % 3)
        return 0
    lax.fori_loop(0, NCH, chunk, 0)
    for par in range(3):
        pltpu.make_async_copy(xr.at[par], accsh.at[cixs.at[par]],
                              ss[par]).wait()

    plsc.subcore_barrier()
    pltpu.sync_copy(accsh.at[pl.ds(s * ZB, ZB)], acc_out.at[c, pl.ds(s * ZB, ZB)])


def _norm_body(x_ref, o_ref):
    xb = x_ref[...]
    ss = jnp.sum(xb * xb, axis=1, keepdims=True)
    o_ref[...] = xb * lax.rsqrt(jnp.maximum(ss, 1e-12))


def _normalize(x):
    return pl.pallas_call(
        _norm_body,
        grid=(10,),
        in_specs=[pl.BlockSpec((N // 10, D), lambda i: (i, 0))],
        out_specs=pl.BlockSpec((N // 10, D), lambda i: (i, 0)),
        out_shape=jax.ShapeDtypeStruct((N, D), jnp.float32),
    )(x)


def _invrow_body(p_ref, o_ref):
    rs = p_ref[0] + p_ref[1]
    o_ref[...] = 1.0 / jnp.maximum(rs, 1e-12)


def _vec_invrow(rsparts):
    return pl.pallas_call(
        _invrow_body,
        out_shape=jax.ShapeDtypeStruct((NPAD // D, D), jnp.float32),
    )(rsparts.reshape(NC, NPAD // D, D))


def _deg_body(p_ref, ir_ref, dv_ref, dr_ref):
    deg = p_ref[0] + p_ref[1] + 1.0
    dv = lax.rsqrt(deg)
    dv_ref[...] = dv
    dr_ref[...] = dv * ir_ref[...]


def _vec_deg(degparts, invrow):
    shp = jax.ShapeDtypeStruct((NPAD // D, D), jnp.float32)
    return pl.pallas_call(
        _deg_body,
        out_shape=(shp, shp),
    )(degparts.reshape(NC, NPAD // D, D), invrow)


def _mm_body(a0_ref, a1_ref, x_ref, dv_ref, w_ref, b_ref, o_ref, *, relu):
    dv = dv_ref[...]
    a = (a0_ref[...] + a1_ref[...] + x_ref[...] * dv) * dv
    h = jnp.dot(a, w_ref[...], preferred_element_type=jnp.float32) + b_ref[...]
    o_ref[...] = jnp.maximum(h, 0.0) if relu else h


def _mm(a0, a1, xpad, sw, w, b, relu):
    blk = NPAD // 10
    return pl.pallas_call(
        functools.partial(_mm_body, relu=relu),
        grid=(10,),
        in_specs=[
            pl.BlockSpec((blk, D), lambda i: (i, 0)),
            pl.BlockSpec((blk, D), lambda i: (i, 0)),
            pl.BlockSpec((blk, D), lambda i: (i, 0)),
            pl.BlockSpec((blk, 1), lambda i: (i, 0)),
            pl.BlockSpec((D, D), lambda i: (0, 0)),
            pl.BlockSpec((1, D), lambda i: (0, 0)),
        ],
        out_specs=pl.BlockSpec((blk, D), lambda i: (i, 0)),
        out_shape=jax.ShapeDtypeStruct((NPAD, D), jnp.float32),
    )(a0, a1, xpad, sw, w, b)


def _layer(x, xn, row3, col3, w_mat, b_vec, relu):
    att3, rsparts = _sc_attention(xn, row3, col3)
    invrow = _vec_invrow(rsparts)
    degparts = _sc_degree(att3, row3, col3, invrow.reshape(NPAD))
    dinv, dr = _vec_deg(degparts, invrow)
    accparts = _sc_aggregate(x, att3, row3, col3, dr.reshape(NPAD))
    xpad = jnp.pad(x, ((0, NPAD - N), (0, 0)))
    h = _mm(accparts[0], accparts[1], xpad, dinv.reshape(NPAD, 1),
            w_mat, b_vec.reshape(1, D), relu)
    return h[:N]


def kernel(x, edge_index, W1, b1, W2, b2):
    row3 = edge_index[0].astype(jnp.int32).reshape(NW, NCH, CH)
    col3 = edge_index[1].astype(jnp.int32).reshape(NW, NCH, CH)
    xn = _normalize(x)
    h1 = _layer(x, xn, row3, col3, W1, b1, relu=True)
    out = _layer(h1, _normalize(h1), row3, col3, W2, b2, relu=False)
    return out
